# baseline (speedup 1.0000x reference)
import functools

import jax
import jax.numpy as jnp
from jax import lax
from jax.experimental import pallas as pl
from jax.experimental.pallas import tpu as pltpu
from jax.experimental.pallas import tpu_sc as plsc

D_MODEL = 64
NUM_WORKERS = 32
CHUNK = 128
TOTAL = 4096 * 200
PER_WORKER = TOTAL // NUM_WORKERS
CHUNKS_PER_WORKER = PER_WORKER // CHUNK


def _gather_kernel(table_hbm, idx_hbm, out_hbm, idx_v, rows_v, gsem):
    wid = lax.axis_index("s") * 2 + lax.axis_index("c")

    pltpu.sync_copy(idx_hbm.at[wid], idx_v)

    def body(j, carry):
        pltpu.async_copy(table_hbm.at[idx_v.at[j]], rows_v, gsem).wait()
        pltpu.sync_copy(rows_v, out_hbm.at[wid, j])
        return carry

    lax.fori_loop(0, CHUNKS_PER_WORKER, body, 0, unroll=False)


@jax.jit
def _embed(x_blocked, table):
    mesh = plsc.VectorSubcoreMesh(core_axis_name="c", subcore_axis_name="s")
    run = pl.kernel(
        _gather_kernel,
        out_type=jax.ShapeDtypeStruct(
            (NUM_WORKERS, CHUNKS_PER_WORKER, CHUNK, D_MODEL), jnp.float32
        ),
        mesh=mesh,
        scratch_types=[
            pltpu.VMEM((CHUNKS_PER_WORKER, CHUNK), jnp.int32),
            pltpu.VMEM((CHUNK, D_MODEL), jnp.float32),
            pltpu.SemaphoreType.DMA,
        ],
        compiler_params=pltpu.CompilerParams(use_tc_tiling_on_sc=False),
    )
    return run(table, x_blocked)


def kernel(x, table):
    b, t = x.shape
    x_blocked = x.reshape(NUM_WORKERS, CHUNKS_PER_WORKER, CHUNK).astype(jnp.int32)
    out = _embed(x_blocked, table)
    return out.reshape(b, t, D_MODEL)


# speedup vs baseline: 1.1109x; 1.1109x over previous
import jax
import jax.numpy as jnp
from jax import lax
from jax.experimental import pallas as pl
from jax.experimental.pallas import tpu as pltpu
from jax.experimental.pallas import tpu_sc as plsc

D_MODEL = 64
NUM_WORKERS = 32
LANE = 128
BIG = 2
NBUF = 4
TOTAL = 4096 * 200
PER_WORKER = TOTAL // NUM_WORKERS
IDX_ROWS = PER_WORKER // LANE
NB = IDX_ROWS // BIG


def _gather_kernel(table_hbm, idx_hbm, out_hbm, idx_v,
                   r0, r1, r2, r3, g0, g1, g2, g3, s0, s1, s2, s3):
    wid = lax.axis_index("s") * 2 + lax.axis_index("c")
    rows = [r0, r1, r2, r3]
    gsem = [g0, g1, g2, g3]
    ssem = [s0, s1, s2, s3]

    pltpu.sync_copy(idx_hbm.at[wid], idx_v)

    def fire_gathers(chunk, b):
        for r in range(BIG):
            pltpu.async_copy(
                table_hbm.at[idx_v.at[BIG * chunk + r]], rows[b].at[r], gsem[b]
            )

    def wait_gathers(chunk, b):
        pltpu.make_async_copy(out_hbm.at[wid, chunk], rows[b], gsem[b]).wait()

    def fire_store(chunk, b):
        pltpu.async_copy(rows[b], out_hbm.at[wid, chunk], ssem[b])

    def wait_store(chunk, b):
        pltpu.make_async_copy(rows[b], out_hbm.at[wid, chunk], ssem[b]).wait()

    def step(chunk, i, refill, fresh):
        wait_gathers(chunk, i)
        fire_store(chunk, i)
        bn = (i + 3) % NBUF
        if refill:
            if not fresh:
                wait_store(chunk, bn)
            fire_gathers(chunk + 3, bn)

    for b in range(3):
        fire_gathers(b, b)

    step(0, 0, True, True)
    step(1, 1, True, False)
    step(2, 2, True, False)
    step(3, 3, True, False)

    def body(t, carry):
        for i in range(NBUF):
            chunk = NBUF * t + i
            wait_gathers(chunk, i)
            fire_store(chunk, i)
            bn = (i + 3) % NBUF
            wait_store(chunk, bn)
            fire_gathers(chunk + 3, bn)
        return carry

    lax.fori_loop(1, NB // NBUF - 1, body, 0, unroll=False)

    base = NB - NBUF
    step(base, 0, True, False)
    step(base + 1, 1, False, False)
    step(base + 2, 2, False, False)
    step(base + 3, 3, False, False)

    for b in range(NBUF):
        wait_store(b, b)


@jax.jit
def _embed(x_blocked, table):
    mesh = plsc.VectorSubcoreMesh(core_axis_name="c", subcore_axis_name="s")
    run = pl.kernel(
        _gather_kernel,
        out_type=jax.ShapeDtypeStruct(
            (NUM_WORKERS, NB, BIG, LANE, D_MODEL), jnp.float32
        ),
        mesh=mesh,
        scratch_types=[
            pltpu.VMEM((IDX_ROWS, LANE), jnp.int32),
            pltpu.VMEM((BIG, LANE, D_MODEL), jnp.float32),
            pltpu.VMEM((BIG, LANE, D_MODEL), jnp.float32),
            pltpu.VMEM((BIG, LANE, D_MODEL), jnp.float32),
            pltpu.VMEM((BIG, LANE, D_MODEL), jnp.float32),
            pltpu.SemaphoreType.DMA,
            pltpu.SemaphoreType.DMA,
            pltpu.SemaphoreType.DMA,
            pltpu.SemaphoreType.DMA,
            pltpu.SemaphoreType.DMA,
            pltpu.SemaphoreType.DMA,
            pltpu.SemaphoreType.DMA,
            pltpu.SemaphoreType.DMA,
        ],
        compiler_params=pltpu.CompilerParams(use_tc_tiling_on_sc=False),
    )
    return run(table, x_blocked)


def kernel(x, table):
    b, t = x.shape
    x_blocked = x.reshape(NUM_WORKERS, IDX_ROWS, LANE).astype(jnp.int32)
    out = _embed(x_blocked, table)
    return out.reshape(b, t, D_MODEL)
